# baseline (speedup 1.0000x reference)
import functools

import jax
import jax.numpy as jnp
from jax import lax
from jax.experimental import pallas as pl
from jax.experimental.pallas import tpu as pltpu
from jax.experimental.pallas import tpu_sc as plsc

F = 128
FH = F // 2
CHUNK = 128
NCORES = 2
NSUB = 16
NBUF = 5
ZR = 128
BLK = 1000


@functools.partial(jax.jit, static_argnames=("n_nodes", "n_chunks", "phases"))
def _gather_sum(u4, col3, row3, zsrc, *, n_nodes, n_chunks, phases):
    acc_rows = ((n_nodes + NSUB * ZR) // (NSUB * ZR)) * NSUB * ZR
    zcopies = acc_rows // NSUB // ZR
    drows = n_nodes // NSUB
    assert drows * NSUB == n_nodes

    mesh = plsc.VectorSubcoreMesh(core_axis_name="c", subcore_axis_name="s",
                                  num_cores=NCORES, num_subcores=NSUB)

    @functools.partial(
        pl.kernel,
        out_type=jax.ShapeDtypeStruct((phases, NCORES, NSUB, drows, FH),
                                      jnp.float32),
        mesh=mesh,
        compiler_params=pltpu.CompilerParams(use_tc_tiling_on_sc=False),
        scratch_types=[
            pltpu.VMEM((n_chunks, CHUNK), jnp.int32),
            pltpu.VMEM((n_chunks, CHUNK), jnp.int32),
        ] + [pltpu.VMEM((CHUNK, FH), jnp.float32) for _ in range(NBUF)]
          + [pltpu.VMEM_SHARED((acc_rows, FH), jnp.float32)]
          + [pltpu.SemaphoreType.DMA for _ in range(NBUF)],
    )
    def k(u_hbm, col_hbm, row_hbm, z_hbm, out_hbm, col_v, row_v,
          b0, b1, b2, b3, b4, acc, s0, s1, s2, s3, s4):
        bufs = [b0, b1, b2, b3, b4]
        sems = [s0, s1, s2, s3, s4]
        cid = lax.axis_index("c")
        sid = lax.axis_index("s")

        pltpu.sync_copy(col_hbm.at[sid], col_v)
        pltpu.sync_copy(row_hbm.at[sid], row_v)

        def phase_body(p):
            for j in range(zcopies):
                pltpu.sync_copy(
                    z_hbm, acc.at[pl.ds((sid * zcopies + j) * ZR, ZR)])
            plsc.subcore_barrier()

            my_u = u_hbm.at[p, cid]

            descs = [
                pltpu.async_copy(my_u.at[col_v.at[b]], bufs[b], sems[b])
                for b in range(min(NBUF, n_chunks))
            ]
            for c in range(n_chunks):
                b = c % NBUF
                descs[b].wait()
                pltpu.sync_copy(bufs[b], acc.at[row_v.at[c]], add=True)
                if c + NBUF < n_chunks:
                    descs[b] = pltpu.async_copy(
                        my_u.at[col_v.at[c + NBUF]], bufs[b], sems[b])

            plsc.subcore_barrier()
            pltpu.sync_copy(acc.at[pl.ds(sid * drows, drows)],
                            out_hbm.at[p, cid, sid])
            plsc.subcore_barrier()

        if phases == 1:
            phase_body(0)
        else:
            pl.loop(0, phases)(phase_body)

    return k(u4, col3, row3, zsrc).reshape(phases, NCORES, n_nodes, FH)


def _dis_from_deg(deg2, n_nodes):
    nb = n_nodes // BLK

    def body(p_ref, dis_ref):
        deg = p_ref[0, :, 0:1]
        dis_ref[...] = jnp.where(deg > 0.0,
                                 1.0 / jnp.sqrt(jnp.maximum(deg, 1.0)), 0.0)

    return pl.pallas_call(
        body,
        grid=(nb,),
        in_specs=[pl.BlockSpec((1, BLK, FH), lambda i: (0, i, 0))],
        out_specs=pl.BlockSpec((BLK, 1), lambda i: (i, 0)),
        out_shape=jax.ShapeDtypeStruct((n_nodes, 1), jnp.float32),
    )(deg2)


def _split(v):
    return jnp.stack([v[:, :FH], v[:, FH:]])


def _merge(p_ref):
    return jnp.concatenate([p_ref[0], p_ref[1]], axis=1)


def _scale_all(x_seq, dis, t_steps, n_nodes):
    nb = n_nodes // BLK

    def body(x_ref, d_ref, u_ref):
        u_ref[...] = _split(x_ref[0] * d_ref[...])[None]

    return pl.pallas_call(
        body,
        grid=(t_steps * nb,),
        in_specs=[
            pl.BlockSpec((1, BLK, F), lambda i: (i // nb, i % nb, 0)),
            pl.BlockSpec((BLK, 1), lambda i: (i % nb, 0)),
        ],
        out_specs=pl.BlockSpec((1, NCORES, BLK, FH),
                               lambda i: (i // nb, 0, i % nb, 0)),
        out_shape=jax.ShapeDtypeStruct((t_steps, NCORES, n_nodes, FH),
                                       jnp.float32),
    )(x_seq, dis)


def _xpre_all(x_seq, px, dis, Wxz, bxz, Wxr, bxr, Wxh, bxh, t_steps,
              n_nodes):
    nb = n_nodes // BLK

    def body(x_ref, p_ref, d_ref, wz_ref, bz_ref, wr_ref, br_ref,
             wh_ref, bh_ref, xz_ref, xr_ref, xh_ref):
        x = x_ref[0]
        s = d_ref[...] * _merge(p_ref[0])
        for w_ref, b_ref, o_ref in ((wz_ref, bz_ref, xz_ref),
                                    (wr_ref, br_ref, xr_ref),
                                    (wh_ref, bh_ref, xh_ref)):
            o_ref[...] = (jnp.dot(x, w_ref[0], preferred_element_type=jnp.float32)
                          - jnp.dot(s, w_ref[1], preferred_element_type=jnp.float32)
                          + b_ref[...])[None]

    wspec = pl.BlockSpec((2, F, F), lambda i: (0, 0, 0))
    bspec = pl.BlockSpec((1, F), lambda i: (0, 0))
    ospec = pl.BlockSpec((1, BLK, F), lambda i: (i // nb, i % nb, 0))
    oshape = jax.ShapeDtypeStruct((t_steps, n_nodes, F), jnp.float32)
    return pl.pallas_call(
        body,
        grid=(t_steps * nb,),
        in_specs=[
            pl.BlockSpec((1, BLK, F), lambda i: (i // nb, i % nb, 0)),
            pl.BlockSpec((1, NCORES, BLK, FH),
                         lambda i: (i // nb, 0, i % nb, 0)),
            pl.BlockSpec((BLK, 1), lambda i: (i % nb, 0)),
            wspec, bspec, wspec, bspec, wspec, bspec,
        ],
        out_specs=[ospec, ospec, ospec],
        out_shape=[oshape, oshape, oshape],
    )(x_seq, px, dis, Wxz, bxz.reshape(1, F), Wxr, bxr.reshape(1, F),
      Wxh, bxh.reshape(1, F))


def _gates(h, ph, xz_all, xr_all, dis, Whz, bhz, Whr, bhr, t, n_nodes):
    nb = n_nodes // BLK

    def body(h_ref, p_ref, xz_ref, xr_ref, d_ref, wz_ref, bz_ref,
             wr_ref, br_ref, z_ref, s_ref, us_ref):
        h_ = h_ref[...]
        s_ = d_ref[...] * _merge(p_ref)
        z = jax.nn.sigmoid(
            xz_ref[0] + jnp.dot(h_, wz_ref[0], preferred_element_type=jnp.float32)
            - jnp.dot(s_, wz_ref[1], preferred_element_type=jnp.float32)
            + bz_ref[...])
        r = jax.nn.sigmoid(
            xr_ref[0] + jnp.dot(h_, wr_ref[0], preferred_element_type=jnp.float32)
            - jnp.dot(s_, wr_ref[1], preferred_element_type=jnp.float32)
            + br_ref[...])
        hr = h_ * r
        z_ref[...] = z
        s_ref[...] = hr
        us_ref[...] = _split(d_ref[...] * hr)

    bs = pl.BlockSpec((BLK, F), lambda i: (i, 0))
    ts = pl.BlockSpec((1, BLK, F), lambda i: (t, i, 0))
    wspec = pl.BlockSpec((2, F, F), lambda i: (0, 0, 0))
    bspec = pl.BlockSpec((1, F), lambda i: (0, 0))
    oshape = jax.ShapeDtypeStruct((n_nodes, F), jnp.float32)
    return pl.pallas_call(
        body,
        grid=(nb,),
        in_specs=[
            bs,
            pl.BlockSpec((NCORES, BLK, FH), lambda i: (0, i, 0)),
            ts, ts,
            pl.BlockSpec((BLK, 1), lambda i: (i, 0)),
            wspec, bspec, wspec, bspec,
        ],
        out_specs=[bs, bs,
                   pl.BlockSpec((NCORES, BLK, FH), lambda i: (0, i, 0))],
        out_shape=[oshape, oshape,
                   jax.ShapeDtypeStruct((NCORES, n_nodes, FH), jnp.float32)],
    )(h, ph, xz_all, xr_all, dis, Whz, bhz.reshape(1, F), Whr,
      bhr.reshape(1, F))


def _update(h, z, s, ps, xh_all, dis, Whh, bhh, W_lin, b_lin, t, n_nodes):
    nb = n_nodes // BLK

    def body(h_ref, z_ref, s_ref, p_ref, xh_ref, d_ref, wh_ref, bh_ref,
             wl_ref, bl_ref, hn_ref, uh_ref, y_ref):
        s_ = d_ref[...] * _merge(p_ref)
        ht = jnp.tanh(
            xh_ref[0]
            + jnp.dot(s_ref[...], wh_ref[0], preferred_element_type=jnp.float32)
            - jnp.dot(s_, wh_ref[1], preferred_element_type=jnp.float32)
            + bh_ref[...])
        z = z_ref[...]
        hn = z * h_ref[...] + (1.0 - z) * ht
        hn_ref[...] = hn
        uh_ref[...] = _split(d_ref[...] * hn)
        y_ref[...] = (jnp.dot(hn, wl_ref[...], preferred_element_type=jnp.float32)
                      + bl_ref[...])

    bs = pl.BlockSpec((BLK, F), lambda i: (i, 0))
    return pl.pallas_call(
        body,
        grid=(nb,),
        in_specs=[
            bs, bs, bs,
            pl.BlockSpec((NCORES, BLK, FH), lambda i: (0, i, 0)),
            pl.BlockSpec((1, BLK, F), lambda i: (t, i, 0)),
            pl.BlockSpec((BLK, 1), lambda i: (i, 0)),
            pl.BlockSpec((2, F, F), lambda i: (0, 0, 0)),
            pl.BlockSpec((1, F), lambda i: (0, 0)),
            pl.BlockSpec((F, 1), lambda i: (0, 0)),
            pl.BlockSpec((1, 1), lambda i: (0, 0)),
        ],
        out_specs=[bs, pl.BlockSpec((NCORES, BLK, FH), lambda i: (0, i, 0)),
                   pl.BlockSpec((BLK, 1), lambda i: (i, 0))],
        out_shape=[
            jax.ShapeDtypeStruct((n_nodes, F), jnp.float32),
            jax.ShapeDtypeStruct((NCORES, n_nodes, FH), jnp.float32),
            jax.ShapeDtypeStruct((n_nodes, 1), jnp.float32),
        ],
    )(h, z, s, ps, xh_all, dis, Whh, bhh.reshape(1, F), W_lin,
      b_lin.reshape(1, 1))


def kernel(x_seq, edge_index, Wxz, bxz, Whz, bhz, Wxr, bxr, Whr, bhr,
           Wxh, bxh, Whh, bhh, W_lin, b_lin):
    T, n_nodes, _ = x_seq.shape
    e = edge_index.shape[1]

    ept = -(-e // NSUB)
    n_chunks = -(-ept // CHUNK)
    ep = NSUB * n_chunks * CHUNK
    row = edge_index[0]
    col = edge_index[1]
    pad = ep - e
    colp = jnp.concatenate([col, jnp.zeros((pad,), jnp.int32)])
    rowp = jnp.concatenate([row, jnp.full((pad,), n_nodes, jnp.int32)])
    col3 = colp.reshape(NSUB, n_chunks, CHUNK)
    row3 = rowp.reshape(NSUB, n_chunks, CHUNK)
    zsrc = jnp.zeros((ZR, FH), jnp.float32)

    def gs(u4, phases):
        return _gather_sum(u4, col3, row3, zsrc, n_nodes=n_nodes,
                           n_chunks=n_chunks, phases=phases)

    deg2 = gs(jnp.ones((1, NCORES, n_nodes, FH), jnp.float32), 1)[0]
    dis = _dis_from_deg(deg2, n_nodes)

    u_all = _scale_all(x_seq, dis, T, n_nodes)
    px_all = jnp.stack([gs(u_all[t][None], 1)[0] for t in range(T)])
    xz_all, xr_all, xh_all = _xpre_all(x_seq, px_all, dis, Wxz, bxz,
                                       Wxr, bxr, Wxh, bxh, T, n_nodes)

    zerosP = jnp.zeros((NCORES, n_nodes, FH), jnp.float32)
    h = jnp.zeros((n_nodes, F), jnp.float32)
    uh = None
    ys = []
    for t in range(T):
        ph = zerosP if t == 0 else gs(uh[None], 1)[0]
        z, s, us = _gates(h, ph, xz_all, xr_all, dis, Whz, bhz, Whr, bhr,
                          t, n_nodes)
        ps = zerosP if t == 0 else gs(us[None], 1)[0]
        h, uh, y = _update(h, z, s, ps, xh_all, dis, Whh, bhh, W_lin, b_lin,
                           t, n_nodes)
        ys.append(y)
    return jnp.stack(ys, axis=0)


# speedup vs baseline: 1.1152x; 1.1152x over previous
import functools

import jax
import jax.numpy as jnp
from jax import lax
from jax.experimental import pallas as pl
from jax.experimental.pallas import tpu as pltpu
from jax.experimental.pallas import tpu_sc as plsc

F = 128
FH = F // 2
CHUNK = 128
NCORES = 2
NSUB = 16
NBUF = 5
ZR = 128
BLK = 1000


@functools.partial(jax.jit, static_argnames=("n_nodes", "n_chunks", "phases"))
def _gather_sum(u4, col3, row3, zsrc, *, n_nodes, n_chunks, phases):
    acc_rows = ((n_nodes + NSUB * ZR) // (NSUB * ZR)) * NSUB * ZR
    zcopies = acc_rows // NSUB // ZR
    drows = n_nodes // NSUB
    assert drows * NSUB == n_nodes

    mesh = plsc.VectorSubcoreMesh(core_axis_name="c", subcore_axis_name="s",
                                  num_cores=NCORES, num_subcores=NSUB)

    @functools.partial(
        pl.kernel,
        out_type=jax.ShapeDtypeStruct((phases, NCORES, NSUB, drows, FH),
                                      jnp.float32),
        mesh=mesh,
        compiler_params=pltpu.CompilerParams(use_tc_tiling_on_sc=False),
        scratch_types=[
            pltpu.VMEM((n_chunks, CHUNK), jnp.int32),
            pltpu.VMEM((n_chunks, CHUNK), jnp.int32),
        ] + [pltpu.VMEM((CHUNK, FH), jnp.float32) for _ in range(NBUF)]
          + [pltpu.VMEM_SHARED((acc_rows, FH), jnp.float32)]
          + [pltpu.SemaphoreType.DMA for _ in range(NBUF)],
    )
    def k(u_hbm, col_hbm, row_hbm, z_hbm, out_hbm, col_v, row_v,
          b0, b1, b2, b3, b4, acc, s0, s1, s2, s3, s4):
        bufs = [b0, b1, b2, b3, b4]
        sems = [s0, s1, s2, s3, s4]
        cid = lax.axis_index("c")
        sid = lax.axis_index("s")

        pltpu.sync_copy(col_hbm.at[sid], col_v)
        pltpu.sync_copy(row_hbm.at[sid], row_v)

        def phase_body(p):
            for j in range(zcopies):
                pltpu.sync_copy(
                    z_hbm, acc.at[pl.ds((sid * zcopies + j) * ZR, ZR)])
            plsc.subcore_barrier()

            my_u = u_hbm.at[p, cid]

            descs = [
                pltpu.async_copy(my_u.at[col_v.at[b]], bufs[b], sems[b])
                for b in range(min(NBUF, n_chunks))
            ]
            for c in range(n_chunks):
                b = c % NBUF
                descs[b].wait()
                pltpu.sync_copy(bufs[b], acc.at[row_v.at[c]], add=True)
                if c + NBUF < n_chunks:
                    descs[b] = pltpu.async_copy(
                        my_u.at[col_v.at[c + NBUF]], bufs[b], sems[b])

            plsc.subcore_barrier()
            pltpu.sync_copy(acc.at[pl.ds(sid * drows, drows)],
                            out_hbm.at[p, cid, sid])
            if phases > 1:
                plsc.subcore_barrier()

        if phases == 1:
            phase_body(0)
        else:
            pl.loop(0, phases)(phase_body)

    return k(u4, col3, row3, zsrc).reshape(phases, NCORES, n_nodes, FH)


def _dis_from_deg(deg2, n_nodes):
    nb = n_nodes // BLK

    def body(p_ref, dis_ref):
        deg = p_ref[0, :, 0:1]
        dis_ref[...] = jnp.where(deg > 0.0,
                                 1.0 / jnp.sqrt(jnp.maximum(deg, 1.0)), 0.0)

    return pl.pallas_call(
        body,
        grid=(nb,),
        in_specs=[pl.BlockSpec((1, BLK, FH), lambda i: (0, i, 0))],
        out_specs=pl.BlockSpec((BLK, 1), lambda i: (i, 0)),
        out_shape=jax.ShapeDtypeStruct((n_nodes, 1), jnp.float32),
    )(deg2)


def _split(v):
    return jnp.stack([v[:, :FH], v[:, FH:]])


def _merge(p_ref):
    return jnp.concatenate([p_ref[0], p_ref[1]], axis=1)


def _scale(x_seq, dis, t, n_nodes):
    nb = n_nodes // BLK

    def body(x_ref, d_ref, u_ref):
        u_ref[...] = _split(x_ref[0] * d_ref[...])

    return pl.pallas_call(
        body,
        grid=(nb,),
        in_specs=[
            pl.BlockSpec((1, BLK, F), lambda i: (t, i, 0)),
            pl.BlockSpec((BLK, 1), lambda i: (i, 0)),
        ],
        out_specs=pl.BlockSpec((NCORES, BLK, FH), lambda i: (0, i, 0)),
        out_shape=jax.ShapeDtypeStruct((NCORES, n_nodes, FH), jnp.float32),
    )(x_seq, dis)


def _xpre(x_seq, px, dis, Wxz, bxz, Wxr, bxr, Wxh, bxh, t, n_nodes):
    nb = n_nodes // BLK

    def body(x_ref, p_ref, d_ref, wz_ref, bz_ref, wr_ref, br_ref,
             wh_ref, bh_ref, xz_ref, xr_ref, xh_ref):
        x = x_ref[0]
        s = d_ref[...] * _merge(p_ref)
        for w_ref, b_ref, o_ref in ((wz_ref, bz_ref, xz_ref),
                                    (wr_ref, br_ref, xr_ref),
                                    (wh_ref, bh_ref, xh_ref)):
            o_ref[...] = (jnp.dot(x, w_ref[0], preferred_element_type=jnp.float32)
                          - jnp.dot(s, w_ref[1], preferred_element_type=jnp.float32)
                          + b_ref[...])

    wspec = pl.BlockSpec((2, F, F), lambda i: (0, 0, 0))
    bspec = pl.BlockSpec((1, F), lambda i: (0, 0))
    ospec = pl.BlockSpec((BLK, F), lambda i: (i, 0))
    oshape = jax.ShapeDtypeStruct((n_nodes, F), jnp.float32)
    return pl.pallas_call(
        body,
        grid=(nb,),
        in_specs=[
            pl.BlockSpec((1, BLK, F), lambda i: (t, i, 0)),
            pl.BlockSpec((NCORES, BLK, FH), lambda i: (0, i, 0)),
            pl.BlockSpec((BLK, 1), lambda i: (i, 0)),
            wspec, bspec, wspec, bspec, wspec, bspec,
        ],
        out_specs=[ospec, ospec, ospec],
        out_shape=[oshape, oshape, oshape],
    )(x_seq, px, dis, Wxz, bxz.reshape(1, F), Wxr, bxr.reshape(1, F),
      Wxh, bxh.reshape(1, F))


def _gates(h, ph, xz, xr, dis, Whz, bhz, Whr, bhr, n_nodes):
    nb = n_nodes // BLK

    def body(h_ref, p_ref, xz_ref, xr_ref, d_ref, wz_ref, bz_ref,
             wr_ref, br_ref, z_ref, s_ref, us_ref):
        h_ = h_ref[...]
        s_ = d_ref[...] * _merge(p_ref)
        z = jax.nn.sigmoid(
            xz_ref[...] + jnp.dot(h_, wz_ref[0], preferred_element_type=jnp.float32)
            - jnp.dot(s_, wz_ref[1], preferred_element_type=jnp.float32)
            + bz_ref[...])
        r = jax.nn.sigmoid(
            xr_ref[...] + jnp.dot(h_, wr_ref[0], preferred_element_type=jnp.float32)
            - jnp.dot(s_, wr_ref[1], preferred_element_type=jnp.float32)
            + br_ref[...])
        hr = h_ * r
        z_ref[...] = z
        s_ref[...] = hr
        us_ref[...] = _split(d_ref[...] * hr)

    bs = pl.BlockSpec((BLK, F), lambda i: (i, 0))
    wspec = pl.BlockSpec((2, F, F), lambda i: (0, 0, 0))
    bspec = pl.BlockSpec((1, F), lambda i: (0, 0))
    oshape = jax.ShapeDtypeStruct((n_nodes, F), jnp.float32)
    return pl.pallas_call(
        body,
        grid=(nb,),
        in_specs=[
            bs,
            pl.BlockSpec((NCORES, BLK, FH), lambda i: (0, i, 0)),
            bs, bs,
            pl.BlockSpec((BLK, 1), lambda i: (i, 0)),
            wspec, bspec, wspec, bspec,
        ],
        out_specs=[bs, bs,
                   pl.BlockSpec((NCORES, BLK, FH), lambda i: (0, i, 0))],
        out_shape=[oshape, oshape,
                   jax.ShapeDtypeStruct((NCORES, n_nodes, FH), jnp.float32)],
    )(h, ph, xz, xr, dis, Whz, bhz.reshape(1, F), Whr, bhr.reshape(1, F))


def _update(h, z, s, ps, xh, dis, Whh, bhh, W_lin, b_lin, n_nodes):
    nb = n_nodes // BLK

    def body(h_ref, z_ref, s_ref, p_ref, xh_ref, d_ref, wh_ref, bh_ref,
             wl_ref, bl_ref, hn_ref, uh_ref, y_ref):
        s_ = d_ref[...] * _merge(p_ref)
        ht = jnp.tanh(
            xh_ref[...]
            + jnp.dot(s_ref[...], wh_ref[0], preferred_element_type=jnp.float32)
            - jnp.dot(s_, wh_ref[1], preferred_element_type=jnp.float32)
            + bh_ref[...])
        z = z_ref[...]
        hn = z * h_ref[...] + (1.0 - z) * ht
        hn_ref[...] = hn
        uh_ref[...] = _split(d_ref[...] * hn)
        y_ref[...] = (jnp.dot(hn, wl_ref[...], preferred_element_type=jnp.float32)
                      + bl_ref[...])

    bs = pl.BlockSpec((BLK, F), lambda i: (i, 0))
    return pl.pallas_call(
        body,
        grid=(nb,),
        in_specs=[
            bs, bs, bs,
            pl.BlockSpec((NCORES, BLK, FH), lambda i: (0, i, 0)),
            bs,
            pl.BlockSpec((BLK, 1), lambda i: (i, 0)),
            pl.BlockSpec((2, F, F), lambda i: (0, 0, 0)),
            pl.BlockSpec((1, F), lambda i: (0, 0)),
            pl.BlockSpec((F, 1), lambda i: (0, 0)),
            pl.BlockSpec((1, 1), lambda i: (0, 0)),
        ],
        out_specs=[bs, pl.BlockSpec((NCORES, BLK, FH), lambda i: (0, i, 0)),
                   pl.BlockSpec((BLK, 1), lambda i: (i, 0))],
        out_shape=[
            jax.ShapeDtypeStruct((n_nodes, F), jnp.float32),
            jax.ShapeDtypeStruct((NCORES, n_nodes, FH), jnp.float32),
            jax.ShapeDtypeStruct((n_nodes, 1), jnp.float32),
        ],
    )(h, z, s, ps, xh, dis, Whh, bhh.reshape(1, F), W_lin,
      b_lin.reshape(1, 1))


def kernel(x_seq, edge_index, Wxz, bxz, Whz, bhz, Wxr, bxr, Whr, bhr,
           Wxh, bxh, Whh, bhh, W_lin, b_lin):
    T, n_nodes, _ = x_seq.shape
    e = edge_index.shape[1]

    ept = -(-e // NSUB)
    n_chunks = -(-ept // CHUNK)
    ep = NSUB * n_chunks * CHUNK
    row = edge_index[0]
    col = edge_index[1]
    pad = ep - e
    colp = jnp.concatenate([col, jnp.zeros((pad,), jnp.int32)])
    rowp = jnp.concatenate([row, jnp.full((pad,), n_nodes, jnp.int32)])
    col3 = colp.reshape(NSUB, n_chunks, CHUNK)
    row3 = rowp.reshape(NSUB, n_chunks, CHUNK)
    zsrc = jnp.zeros((ZR, FH), jnp.float32)

    def gs(u4, phases):
        return _gather_sum(u4, col3, row3, zsrc, n_nodes=n_nodes,
                           n_chunks=n_chunks, phases=phases)

    deg2 = gs(jnp.ones((1, NCORES, n_nodes, FH), jnp.float32), 1)[0]
    dis = _dis_from_deg(deg2, n_nodes)

    xzs, xrs, xhs = [], [], []
    for t in range(T):
        u_t = _scale(x_seq, dis, t, n_nodes)
        px_t = gs(u_t[None], 1)[0]
        xz, xr, xh = _xpre(x_seq, px_t, dis, Wxz, bxz, Wxr, bxr, Wxh, bxh,
                           t, n_nodes)
        xzs.append(xz)
        xrs.append(xr)
        xhs.append(xh)

    zerosP = jnp.zeros((NCORES, n_nodes, FH), jnp.float32)
    h = jnp.zeros((n_nodes, F), jnp.float32)
    uh = None
    ys = []
    for t in range(T):
        ph = zerosP if t == 0 else gs(uh[None], 1)[0]
        z, s, us = _gates(h, ph, xzs[t], xrs[t], dis, Whz, bhz, Whr, bhr,
                          n_nodes)
        ps = zerosP if t == 0 else gs(us[None], 1)[0]
        h, uh, y = _update(h, z, s, ps, xhs[t], dis, Whh, bhh, W_lin, b_lin,
                           n_nodes)
        ys.append(y)
    return jnp.stack(ys, axis=0)
